# Initial kernel scaffold; baseline (speedup 1.0000x reference)
import jax
import jax.numpy as jnp
from jax.experimental import pallas as pl


def kernel(vertex_attr, edgeij_pair, edge_attr, g, batch):
    raise NotImplementedError("write your pallas kernel here")


# speedup vs baseline: 5.5632x; 5.5632x over previous
import functools

import jax
import jax.numpy as jnp
from jax import lax
from jax.experimental import pallas as pl
from jax.experimental.pallas import tpu as pltpu, tpu_sc as plsc

N_V = 10000
N_V_PAD = 10240
E = 320000
D = 16
CHUNK = 80
N_CHUNKS = E // CHUNK
NC, NS = 2, 16
NW = NC * NS
CPW = N_CHUNKS // NW
G = 25
NGROUP = CPW // G
ROWS_PER_TILE = N_V_PAD // NS

_mesh = plsc.VectorSubcoreMesh(core_axis_name="c", subcore_axis_name="s")


@functools.partial(
    pl.kernel,
    out_type=jax.ShapeDtypeStruct((NC, N_V_PAD, D), jnp.float32),
    mesh=_mesh,
    compiler_params=pltpu.CompilerParams(use_tc_tiling_on_sc=False),
    scratch_types=[
        pltpu.VMEM((CPW, CHUNK), jnp.int32),
        pltpu.VMEM((G * CHUNK, D), jnp.float32),
        pltpu.VMEM((ROWS_PER_TILE, D), jnp.float32),
        pltpu.VMEM_SHARED((N_V_PAD, D), jnp.float32),
    ],
)
def _scatter_sc(idx_hbm, edge_hbm, out_hbm, idx_v, edge_v, zbuf, acc):
    c = lax.axis_index("c")
    s = lax.axis_index("s")
    wid = c * NS + s
    row_base = pl.multiple_of(s * ROWS_PER_TILE, ROWS_PER_TILE)

    zero = jnp.zeros((D,), jnp.float32)

    def _zrow(i, carry):
        zbuf[i, :] = zero
        return carry

    lax.fori_loop(0, ROWS_PER_TILE, _zrow, 0)
    pltpu.sync_copy(zbuf, acc.at[pl.ds(row_base, ROWS_PER_TILE)])

    pltpu.sync_copy(idx_hbm.at[wid], idx_v)
    plsc.subcore_barrier()

    def _group(g, carry):
        base_edge = pl.multiple_of(
            (wid * CPW + g * G) * CHUNK, G * CHUNK)
        pltpu.sync_copy(edge_hbm.at[pl.ds(base_edge, G * CHUNK)], edge_v)

        def _chunk(j, carry2):
            pltpu.sync_copy(
                edge_v.at[pl.ds(j * CHUNK, CHUNK)],
                acc.at[idx_v.at[g * G + j]],
                add=True,
            )
            return carry2

        lax.fori_loop(0, G, _chunk, 0)
        return carry

    lax.fori_loop(0, NGROUP, _group, 0)
    plsc.subcore_barrier()

    pltpu.sync_copy(
        acc.at[pl.ds(row_base, ROWS_PER_TILE)],
        out_hbm.at[c].at[pl.ds(row_base, ROWS_PER_TILE)],
    )


def _combine_body(p_ref, o_ref):
    o_ref[...] = p_ref[0] + p_ref[1]


_combine = pl.pallas_call(
    _combine_body,
    out_shape=jax.ShapeDtypeStruct((N_V_PAD * D // 128, 128), jnp.float32),
)


def kernel(vertex_attr, edgeij_pair, edge_attr, g, batch):
    dst = edgeij_pair[1].astype(jnp.int32).reshape(NW, CPW, CHUNK)
    partials = _scatter_sc(dst, edge_attr)
    summed = _combine(partials.reshape(NC, N_V_PAD * D // 128, 128))
    return summed.reshape(N_V_PAD, D)[:N_V]


# speedup vs baseline: 6.0956x; 1.0957x over previous
import functools

import jax
import jax.numpy as jnp
from jax import lax
from jax.experimental import pallas as pl
from jax.experimental.pallas import tpu as pltpu, tpu_sc as plsc

N_V = 10000
N_V_PAD = 10240
E = 320000
D = 16
CHUNK = 125
NC, NS = 2, 16
NW = NC * NS
CPT = E // (NW * CHUNK)
CPG = 16
NGROUP = CPT // CPG
GE = CPG * CHUNK
ROWS_PER_TILE = N_V_PAD // NS

_mesh = plsc.VectorSubcoreMesh(core_axis_name="c", subcore_axis_name="s")


@functools.partial(
    pl.kernel,
    out_type=jax.ShapeDtypeStruct((NC, N_V_PAD, D), jnp.float32),
    mesh=_mesh,
    compiler_params=pltpu.CompilerParams(use_tc_tiling_on_sc=False),
    scratch_types=[
        pltpu.VMEM((CPT, CHUNK), jnp.int32),
        pltpu.VMEM((2, GE, D), jnp.float32),
        pltpu.VMEM((ROWS_PER_TILE, D), jnp.float32),
        pltpu.VMEM_SHARED((N_V_PAD, D), jnp.float32),
        pltpu.SemaphoreType.DMA,
        pltpu.SemaphoreType.DMA,
        pltpu.SemaphoreType.DMA,
        pltpu.SemaphoreType.DMA,
    ],
)
def _scatter_sc(idx_hbm, edge_hbm, out_hbm, idx_v, edge_v, zbuf, acc,
                sem_f0, sem_f1, sem_s0, sem_s1):
    c = lax.axis_index("c")
    s = lax.axis_index("s")
    wid = c * NS + s
    row_base = pl.multiple_of(s * ROWS_PER_TILE, ROWS_PER_TILE)
    sem_f = (sem_f0, sem_f1)
    sem_s = (sem_s0, sem_s1)

    zero = jnp.zeros((D,), jnp.float32)

    def _zrow(i, carry):
        for r in range(8):
            zbuf[i * 8 + r, :] = zero
        return carry

    lax.fori_loop(0, ROWS_PER_TILE // 8, _zrow, 0)
    pltpu.sync_copy(zbuf, acc.at[pl.ds(row_base, ROWS_PER_TILE)])

    pltpu.sync_copy(idx_hbm.at[1].at[wid], idx_v)
    plsc.subcore_barrier()

    def _edge_base(g):
        return pl.multiple_of((wid * CPT + g * CPG) * CHUNK, GE)

    fills = [None, None]
    scats = [[], []]
    fills[0] = pltpu.async_copy(
        edge_hbm.at[pl.ds(_edge_base(0), GE)], edge_v.at[0], sem_f[0])
    for g in range(NGROUP):
        b = g % 2
        nb = (g + 1) % 2
        if g + 1 < NGROUP:
            for d in scats[nb]:
                d.wait()
            scats[nb] = []
            fills[nb] = pltpu.async_copy(
                edge_hbm.at[pl.ds(_edge_base(g + 1), GE)],
                edge_v.at[nb], sem_f[nb])
        fills[b].wait()
        for j in range(CPG):
            scats[b].append(pltpu.async_copy(
                edge_v.at[b].at[pl.ds(j * CHUNK, CHUNK)],
                acc.at[idx_v.at[g * CPG + j]],
                sem_s[b], add=True))
    for b in range(2):
        for d in scats[b]:
            d.wait()
    plsc.subcore_barrier()

    pltpu.sync_copy(
        acc.at[pl.ds(row_base, ROWS_PER_TILE)],
        out_hbm.at[c].at[pl.ds(row_base, ROWS_PER_TILE)],
    )


def _combine_body(p_ref, o_ref):
    o_ref[...] = p_ref[0, : N_V * D // 128] + p_ref[1, : N_V * D // 128]


_combine = pl.pallas_call(
    _combine_body,
    out_shape=jax.ShapeDtypeStruct((N_V * D // 128, 128), jnp.float32),
)


def kernel(vertex_attr, edgeij_pair, edge_attr, g, batch):
    idx = edgeij_pair.astype(jnp.int32).reshape(2, NW, CPT, CHUNK)
    partials = _scatter_sc(idx, edge_attr)
    summed = _combine(partials.reshape(NC, N_V_PAD * D // 128, 128))
    return summed.reshape(N_V, D)


# speedup vs baseline: 6.4668x; 1.0609x over previous
import functools

import jax
import jax.numpy as jnp
import numpy as np
from jax import lax
from jax.experimental import pallas as pl
from jax.experimental.pallas import tpu as pltpu, tpu_sc as plsc

N_V = 10000
N_V_PAD = 10240
E = 320000
D = 16
CHUNK = 128
N_CHUNKS = E // CHUNK
NC, NS = 2, 16
NW = NC * NS
CPT = N_CHUNKS // NW
LEFT = N_CHUNKS - CPT * NW
ROWS_PER_TILE = N_V_PAD // NS

_COL_TABLE = np.tile(np.arange(CHUNK, dtype=np.int32)[:, None], (1, D))

_mesh = plsc.VectorSubcoreMesh(core_axis_name="c", subcore_axis_name="s")


@functools.partial(
    pl.kernel,
    out_type=jax.ShapeDtypeStruct((NC, N_V_PAD, D), jnp.float32),
    mesh=_mesh,
    compiler_params=pltpu.CompilerParams(
        use_tc_tiling_on_sc=False, needs_layout_passes=False),
    scratch_types=[
        pltpu.VMEM((CPT + 1, CHUNK), jnp.int32),
        pltpu.VMEM((CHUNK, D), jnp.int32),
        pltpu.VMEM((2, D, CHUNK), jnp.float32),
        pltpu.VMEM((2, CHUNK, D), jnp.float32),
        pltpu.VMEM((ROWS_PER_TILE, D), jnp.float32),
        pltpu.VMEM_SHARED((N_V_PAD, D), jnp.float32),
        pltpu.SemaphoreType.DMA,
        pltpu.SemaphoreType.DMA,
        pltpu.SemaphoreType.DMA,
        pltpu.SemaphoreType.DMA,
    ],
)
def _scatter_sc(idx_hbm, edge_hbm, col_hbm, out_hbm, idx_v, col_v, stage_v,
                trans_v, zbuf, acc, sem_f0, sem_f1, sem_s0, sem_s1):
    c = lax.axis_index("c")
    s = lax.axis_index("s")
    wid = c * NS + s
    row_base = pl.multiple_of(s * ROWS_PER_TILE, ROWS_PER_TILE)
    sem_f = (sem_f0, sem_f1)
    sem_s = (sem_s0, sem_s1)
    f_iota = lax.iota(jnp.int32, D)
    chunk0 = wid * CPT

    zero = jnp.zeros((D,), jnp.float32)

    def _zrow(i, carry):
        for r in range(8):
            zbuf[i * 8 + r, :] = zero
        return carry

    lax.fori_loop(0, ROWS_PER_TILE // 8, _zrow, 0)
    pltpu.sync_copy(zbuf, acc.at[pl.ds(row_base, ROWS_PER_TILE)])

    pltpu.sync_copy(col_hbm, col_v)
    pltpu.sync_copy(idx_hbm.at[pl.ds(chunk0, CPT), 1, :],
                    idx_v.at[pl.ds(0, CPT)])

    @pl.when(wid < LEFT)
    def _():
        pltpu.sync_copy(idx_hbm.at[NW * CPT + wid, 1, :], idx_v.at[CPT])

    plsc.subcore_barrier()

    def _fill(ec, b):
        pltpu.async_copy(edge_hbm.at[0, ec], stage_v.at[b, 0:8], sem_f[b])
        pltpu.async_copy(edge_hbm.at[1, ec], stage_v.at[b, 8:16], sem_f[b])

    def _wait_fill(b):
        pltpu.make_async_copy(
            edge_hbm.at[0, 0], stage_v.at[b, 0:8], sem_f[b]).wait()
        pltpu.make_async_copy(
            edge_hbm.at[1, 0], stage_v.at[b, 8:16], sem_f[b]).wait()

    def _wait_scat(b):
        pltpu.make_async_copy(
            trans_v.at[b], acc.at[idx_v.at[0]], sem_s[b]).wait()

    def _transpose(b):
        stage_b = stage_v.at[b]

        def _trow(i, carry2):
            for u in range(8):
                e = i * 8 + u
                col = col_v[e, :]
                v = plsc.load_gather(stage_b, [f_iota, col])
                trans_v[b, e, :] = v
            return carry2

        lax.fori_loop(0, CHUNK // 8, _trow, 0)

    _fill(chunk0, 0)

    def _pair(p, carry):
        for b in range(2):
            k = p * 2 + b
            _wait_fill(b)

            @pl.when(k + 1 < CPT)
            def _():
                _fill(chunk0 + k + 1, 1 - b)

            @pl.when(k >= 2)
            def _():
                _wait_scat(b)

            _transpose(b)
            pltpu.async_copy(
                trans_v.at[b], acc.at[idx_v.at[k]], sem_s[b], add=True)
        return carry

    lax.fori_loop(0, CPT // 2, _pair, 0)
    _wait_scat(0)
    _wait_scat(1)

    @pl.when(wid < LEFT)
    def _():
        _fill(NW * CPT + wid, 0)
        _wait_fill(0)
        _transpose(0)
        pltpu.async_copy(
            trans_v.at[0], acc.at[idx_v.at[CPT]], sem_s[0], add=True)
        _wait_scat(0)

    plsc.subcore_barrier()

    pltpu.sync_copy(
        acc.at[pl.ds(row_base, ROWS_PER_TILE)],
        out_hbm.at[c].at[pl.ds(row_base, ROWS_PER_TILE)],
    )


def _combine_body(p_ref, o_ref):
    o_ref[...] = p_ref[0, : N_V * D // 128] + p_ref[1, : N_V * D // 128]


_combine = pl.pallas_call(
    _combine_body,
    out_shape=jax.ShapeDtypeStruct((N_V * D // 128, 128), jnp.float32),
)


def kernel(vertex_attr, edgeij_pair, edge_attr, g, batch):
    idx = (edgeij_pair.astype(jnp.int32)
           .reshape(2, N_CHUNKS, CHUNK).transpose(1, 0, 2))
    edges = (edge_attr.T.reshape(2, 8, N_CHUNKS, CHUNK)
             .transpose(0, 2, 1, 3))
    col_table = jnp.asarray(_COL_TABLE)
    partials = _scatter_sc(idx, edges, col_table)
    summed = _combine(partials.reshape(NC, N_V_PAD * D // 128, 128))
    return summed.reshape(N_V, D)
